# baseline (speedup 1.0000x reference)
import functools

import jax
import jax.numpy as jnp
from jax import lax
from jax.experimental import pallas as pl
from jax.experimental.pallas import tpu as pltpu
from jax.experimental.pallas import tpu_sc as plsc

NW = 32
CHUNK = 128
K = 2


def _wait(src, dst, sem):
    pltpu.make_async_copy(src, dst, sem).wait()


@functools.lru_cache(maxsize=None)
def _build(n_chunk, n_rows, d):
    mesh = plsc.VectorSubcoreMesh(core_axis_name="c", subcore_axis_name="s")
    n_iter = n_chunk // K
    grp_rows = K * CHUNK

    @functools.partial(
        pl.kernel,
        mesh=mesh,
        out_type=jax.ShapeDtypeStruct((NW * n_chunk * CHUNK, d), jnp.float32),
        scratch_types=[
            pltpu.VMEM((n_chunk * CHUNK,), jnp.int32),
            pltpu.VMEM((2, grp_rows, d), jnp.float32),
            pltpu.SemaphoreType.DMA((2,)),
            pltpu.SemaphoreType.DMA((2,)),
        ],
        compiler_params=pltpu.CompilerParams(use_tc_tiling_on_sc=True),
    )
    def emb(ids_hbm, table_hbm, out_hbm, idx_v, rows_v, gsem, ssem):
        wid = lax.axis_index("s") * 2 + lax.axis_index("c")
        base = wid * (n_chunk * CHUNK)
        pltpu.sync_copy(ids_hbm.at[pl.ds(base, n_chunk * CHUNK)], idx_v)

        def gathers(it, grp):
            for b in range(K):
                pltpu.async_copy(
                    table_hbm.at[idx_v.at[pl.ds((it * K + b) * CHUNK, CHUNK)]],
                    rows_v.at[grp, pl.ds(b * CHUNK, CHUNK)],
                    gsem.at[grp],
                )

        def wait_gathers(grp):
            _wait(table_hbm.at[pl.ds(0, grp_rows)], rows_v.at[grp], gsem.at[grp])

        def store(it, grp):
            pltpu.async_copy(
                rows_v.at[grp],
                out_hbm.at[pl.ds(base + it * grp_rows, grp_rows)],
                ssem.at[grp],
            )

        def wait_store(grp):
            _wait(rows_v.at[grp], out_hbm.at[pl.ds(base, grp_rows)], ssem.at[grp])

        gathers(0, 0)
        gathers(1, 1)
        wait_gathers(0)
        store(0, 0)

        def body(g, carry):
            a = g % 2
            bgrp = 1 - a
            wait_store(bgrp)
            gathers(g + 1, bgrp)
            wait_gathers(a)
            store(g, a)
            return carry

        lax.fori_loop(1, n_iter - 1, body, 0)

        a = (n_iter - 1) % 2
        wait_store(1 - a)
        wait_gathers(a)
        store(n_iter - 1, a)
        wait_store(a)

    return emb


def kernel(token_ids, weight):
    b, s = token_ids.shape
    total = b * s
    n_chunk = total // (NW * CHUNK)
    d = weight.shape[1]
    ids = token_ids.reshape(total).astype(jnp.int32)
    out = _build(n_chunk, weight.shape[0], d)(ids, weight)
    return out.reshape(b, s, d)


# speedup vs baseline: 3.1451x; 3.1451x over previous
import functools

import jax
import jax.numpy as jnp
from jax import lax
from jax.experimental import pallas as pl
from jax.experimental.pallas import tpu as pltpu
from jax.experimental.pallas import tpu_sc as plsc

NW = 32
CHUNK = 128
K = 2


def _wait(src, dst, sem):
    pltpu.make_async_copy(src, dst, sem).wait()


@functools.lru_cache(maxsize=None)
def _build(n_chunk, n_rows, d):
    mesh = plsc.VectorSubcoreMesh(core_axis_name="c", subcore_axis_name="s")
    n_iter = n_chunk // K
    grp_rows = K * CHUNK

    @functools.partial(
        pl.kernel,
        mesh=mesh,
        out_type=jax.ShapeDtypeStruct((NW * n_chunk * CHUNK, d), jnp.float32),
        scratch_types=[
            pltpu.VMEM((n_chunk * CHUNK,), jnp.int32),
            pltpu.VMEM((2, grp_rows, d), jnp.float32),
            pltpu.SemaphoreType.DMA((2,)),
            pltpu.SemaphoreType.DMA((2,)),
        ],
        compiler_params=pltpu.CompilerParams(use_tc_tiling_on_sc=True),
    )
    def emb(ids_hbm, table_hbm, out_hbm, idx_v, rows_v, gsem, ssem):
        wid = lax.axis_index("s") * 2 + lax.axis_index("c")
        base = wid * (n_chunk * CHUNK)
        pltpu.sync_copy(ids_hbm.at[pl.ds(base, n_chunk * CHUNK)], idx_v)

        def gathers(it, grp):
            for b in range(K):
                pltpu.async_copy(
                    table_hbm.at[idx_v.at[pl.ds((it * K + b) * CHUNK, CHUNK)]],
                    rows_v.at[grp, pl.ds(b * CHUNK, CHUNK)],
                    gsem.at[grp],
                )

        def wait_gathers(grp):
            _wait(table_hbm.at[pl.ds(0, grp_rows)], rows_v.at[grp], gsem.at[grp])

        def store(it, grp):
            pltpu.async_copy(
                rows_v.at[grp],
                out_hbm.at[pl.ds(base + it * grp_rows, grp_rows)],
                ssem.at[grp],
            )

        def wait_store(grp):
            _wait(rows_v.at[grp], out_hbm.at[pl.ds(base, grp_rows)], ssem.at[grp])

        gathers(0, 0)
        gathers(1, 1)
        wait_gathers(0)
        store(0, 0)

        def body(g, carry):
            a = g % 2
            bgrp = 1 - a
            wait_store(bgrp)
            gathers(g + 1, bgrp)
            wait_gathers(a)
            store(g, a)
            return carry

        lax.fori_loop(1, n_iter - 1, body, 0)

        a = (n_iter - 1) % 2
        wait_store(1 - a)
        wait_gathers(a)
        store(n_iter - 1, a)
        wait_store(a)

    return emb


def kernel(token_ids, weight):
    b, s = token_ids.shape
    total = b * s
    n_chunk = total // (NW * CHUNK)
    d = weight.shape[1]
    ids = token_ids.T.reshape(total).astype(jnp.int32)
    out = _build(n_chunk, weight.shape[0], d)(ids, weight)
    return out.reshape(s, b, d).transpose(1, 0, 2)
